# baseline (speedup 1.0000x reference)
import functools

import jax
import jax.numpy as jnp
from jax import lax
from jax.experimental import pallas as pl
from jax.experimental.pallas import tpu as pltpu
from jax.experimental.pallas import tpu_sc as plsc

_B, _L, _D = 4, 4096, 1024
_NC, _NS = 2, 16
_NW = _NC * _NS
_LPW = _L // _NW
_CH = 8
_NCH = _LPW // _CH
_NVR = _D // 16

_mesh = plsc.VectorSubcoreMesh(
    core_axis_name="c", subcore_axis_name="s",
    num_cores=_NC, num_subcores=_NS,
)


@functools.partial(
    pl.kernel,
    out_type=jax.ShapeDtypeStruct((_B, _L, _D), jnp.float32),
    mesh=_mesh,
    compiler_params=pltpu.CompilerParams(use_tc_tiling_on_sc=True),
    scratch_types=[
        [pltpu.VMEM((_CH, _D), jnp.float32) for _ in range(2)],
        [pltpu.VMEM((_CH, _D), jnp.float32) for _ in range(2 * _B)],
        [pltpu.SemaphoreType.DMA for _ in range(2)],
        [pltpu.SemaphoreType.DMA for _ in range(2 * _B)],
        [pltpu.SemaphoreType.DMA for _ in range(2 * _B)],
    ],
)
def _pos_add(feat_hbm, pos_hbm, out_hbm, pos_v, fb_v, pos_sem, in_sem, out_sem):
    wid = lax.axis_index("s") * _NC + lax.axis_index("c")
    base = wid * _LPW

    def row0(c):
        return base + c * _CH

    def issue_in(c, b, k):
        pltpu.async_copy(
            feat_hbm.at[b, pl.ds(row0(c), _CH), :], fb_v[k * _B + b],
            in_sem[k * _B + b])

    pltpu.async_copy(pos_hbm.at[pl.ds(row0(0), _CH), :], pos_v[0], pos_sem[0])
    pltpu.async_copy(pos_hbm.at[pl.ds(row0(1), _CH), :], pos_v[1], pos_sem[1])
    for b in range(_B):
        issue_in(0, b, 0)
    for b in range(_B):
        issue_in(1, b, 1)

    def half(c2, carry):
        for k in range(2):
            c = c2 * 2 + k
            pltpu.make_async_copy(
                pos_hbm.at[pl.ds(row0(0), _CH), :], pos_v[k],
                pos_sem[k]).wait()
            for b in range(_B):
                s = k * _B + b
                pltpu.make_async_copy(
                    feat_hbm.at[b, pl.ds(row0(0), _CH), :], fb_v[s],
                    in_sem[s]).wait()

                gv, pv = fb_v[s], pos_v[k]

                @plsc.parallel_loop(0, _CH * _NVR, 1, unroll=8)
                def _add(i):
                    r = i >> 6
                    t = (i & (_NVR - 1)) * 16
                    plsc.addupdate(gv.at[r, pl.ds(t, 16)],
                                   pv[r, pl.ds(t, 16)])

                pltpu.async_copy(
                    gv, out_hbm.at[b, pl.ds(row0(c), _CH), :], out_sem[s])

                os = (1 - k) * _B + b
                @pl.when(c + 1 < _NCH)
                def _():
                    @pl.when(c >= 1)
                    def _():
                        pltpu.make_async_copy(
                            fb_v[os],
                            out_hbm.at[b, pl.ds(row0(0), _CH), :],
                            out_sem[os]).wait()
                    issue_in(c + 1, b, 1 - k)

            @pl.when(c + 2 < _NCH)
            def _():
                pltpu.async_copy(
                    pos_hbm.at[pl.ds(row0(c + 2), _CH), :], pos_v[k],
                    pos_sem[k])
        return carry

    lax.fori_loop(0, _NCH // 2, half, 0)

    for b in range(_B):
        for c in (_NCH - 2, _NCH - 1):
            pltpu.make_async_copy(
                fb_v[(c % 2) * _B + b],
                out_hbm.at[b, pl.ds(row0(c), _CH), :],
                out_sem[(c % 2) * _B + b]).wait()


def kernel(feat, pos_emb_table):
    return _pos_add(feat, pos_emb_table)


# speedup vs baseline: 1.0131x; 1.0131x over previous
import functools

import jax
import jax.numpy as jnp
from jax import lax
from jax.experimental import pallas as pl
from jax.experimental.pallas import tpu as pltpu
from jax.experimental.pallas import tpu_sc as plsc

_B, _L, _D = 4, 4096, 1024
_NC, _NS = 2, 16
_NW = _NC * _NS
_LPW = _L // _NW
_CH = 8
_NCH = _LPW // _CH
_NVR = _D // 16

_mesh = plsc.VectorSubcoreMesh(
    core_axis_name="c", subcore_axis_name="s",
    num_cores=_NC, num_subcores=_NS,
)


@functools.partial(
    pl.kernel,
    out_type=jax.ShapeDtypeStruct((_B, _L, _D), jnp.float32),
    mesh=_mesh,
    compiler_params=pltpu.CompilerParams(use_tc_tiling_on_sc=True),
    scratch_types=[
        [pltpu.VMEM((_CH, _D), jnp.float32) for _ in range(2)],
        [pltpu.VMEM((_CH, _D), jnp.float32) for _ in range(2 * _B)],
        [pltpu.SemaphoreType.DMA for _ in range(2)],
        [pltpu.SemaphoreType.DMA for _ in range(2 * _B)],
        [pltpu.SemaphoreType.DMA for _ in range(2 * _B)],
    ],
)
def _pos_add(feat_hbm, pos_hbm, out_hbm, pos_v, fb_v, pos_sem, in_sem, out_sem):
    wid = lax.axis_index("s") * _NC + lax.axis_index("c")
    base = wid * _LPW

    def row0(c):
        return base + c * _CH

    def issue_in(c, b, k):
        pltpu.async_copy(
            feat_hbm.at[b, pl.ds(row0(c), _CH), :], fb_v[k * _B + b],
            in_sem[k * _B + b])

    pltpu.async_copy(pos_hbm.at[pl.ds(row0(0), _CH), :], pos_v[0], pos_sem[0])
    pltpu.async_copy(pos_hbm.at[pl.ds(row0(1), _CH), :], pos_v[1], pos_sem[1])
    for b in range(_B):
        issue_in(0, b, 0)
    for b in range(_B):
        issue_in(1, b, 1)

    def half(c2, carry):
        for k in range(2):
            c = c2 * 2 + k
            pltpu.make_async_copy(
                pos_hbm.at[pl.ds(row0(0), _CH), :], pos_v[k],
                pos_sem[k]).wait()
            for b in range(_B):
                s = k * _B + b
                pltpu.make_async_copy(
                    feat_hbm.at[b, pl.ds(row0(0), _CH), :], fb_v[s],
                    in_sem[s]).wait()

                gv, pv = fb_v[s], pos_v[k]

                @plsc.parallel_loop(0, _CH * _NVR, 1, unroll=8)
                def _add(i):
                    r = i >> 6
                    t = (i & (_NVR - 1)) * 16
                    plsc.addupdate(gv.at[r, pl.ds(t, 16)],
                                   pv[r, pl.ds(t, 16)])

                pltpu.async_copy(
                    gv, out_hbm.at[b, pl.ds(row0(c), _CH), :], out_sem[s])

                os = (1 - k) * _B + b
                @pl.when(jnp.logical_and(c >= 1, c + 1 < _NCH))
                def _():
                    pltpu.make_async_copy(
                        fb_v[os],
                        out_hbm.at[b, pl.ds(row0(0), _CH), :],
                        out_sem[os]).wait()
                    issue_in(c + 1, b, 1 - k)

            @pl.when(c + 2 < _NCH)
            def _():
                pltpu.async_copy(
                    pos_hbm.at[pl.ds(row0(c + 2), _CH), :], pos_v[k],
                    pos_sem[k])
        return carry

    lax.fori_loop(0, _NCH // 2, half, 0)

    for b in range(_B):
        for c in (_NCH - 2, _NCH - 1):
            pltpu.make_async_copy(
                fb_v[(c % 2) * _B + b],
                out_hbm.at[b, pl.ds(row0(c), _CH), :],
                out_sem[(c % 2) * _B + b]).wait()


def kernel(feat, pos_emb_table):
    return _pos_add(feat, pos_emb_table)
